# Initial kernel scaffold; baseline (speedup 1.0000x reference)
import jax
import jax.numpy as jnp
from jax.experimental import pallas as pl


def kernel(x_enc, x_mark_enc, x_dec, x_mark_dec, token_w, time_w, qk_w0, v_w0, out_w0, out_b0, rot0, c1_w0, c1_b0, c2_w0, c2_b0, n1_g0, n1_b0, n2_g0, n2_b0, qk_w1, v_w1, out_w1, out_b1, rot1, c1_w1, c1_b1, c2_w1, c2_b1, n1_g1, n1_b1, n2_g1, n2_b1, norm_g, norm_b, proj_w, proj_b):
    raise NotImplementedError("write your pallas kernel here")


# speedup vs baseline: 3.9627x; 3.9627x over previous
import functools

import jax
import jax.numpy as jnp
from jax import lax
from jax.experimental import pallas as pl
from jax.experimental.pallas import tpu as pltpu
from jax.experimental.pallas import tpu_sc as plsc

B = 2
T = 2048
D = 512
H = 8
DH = 64
DFF = 2048
NHASH = 4
NBUCK = 512
BK = 4
BH = B * H
S = NHASH * T
ROWS = BH * NHASH
ENC_IN = 118
MARK = 6
C_OUT = 118

_INTERPRET = False


def _bdot(a, b):
    return jnp.dot(a.astype(jnp.bfloat16), b.astype(jnp.bfloat16),
                   preferred_element_type=jnp.float32)


def _b16(a):
    return a.astype(jnp.bfloat16).astype(jnp.float32)


def _mm_kern(x_ref, w_ref, *rest, nbias, nres, act, ln):
    i = 0
    refs = list(rest)
    out_ref = refs[-1]
    acc = _bdot(x_ref[...], w_ref[...])
    if nbias:
        acc = acc + refs[i][...]
        i += 1
    if nres:
        acc = acc + refs[i][...]
        i += 1
    if act == "gelu":
        acc = acc * 0.5 * (1.0 + lax.erf(acc * (2.0 ** -0.5)))
    if ln:
        g_ref, b_ref = refs[i], refs[i + 1]
        m = jnp.mean(acc, axis=-1, keepdims=True)
        v = jnp.mean((acc - m) ** 2, axis=-1, keepdims=True)
        acc = (acc - m) / jnp.sqrt(v + 1e-5) * g_ref[...] + b_ref[...]
    out_ref[...] = acc


def _mm(x, w, bias=None, residual=None, act=None, ln=None, bm=512, bn=512):
    M, K = x.shape
    K2, N = w.shape
    assert K == K2
    bm = min(bm, M)
    bn = min(bn, N)
    assert M % bm == 0 and N % bn == 0
    grid = (M // bm, N // bn)
    in_specs = [
        pl.BlockSpec((bm, K), lambda i, j: (i, 0)),
        pl.BlockSpec((K, bn), lambda i, j: (0, j)),
    ]
    args = [x, w]
    if bias is not None:
        in_specs.append(pl.BlockSpec((1, bn), lambda i, j: (0, j)))
        args.append(bias.reshape(1, N))
    if residual is not None:
        in_specs.append(pl.BlockSpec((bm, bn), lambda i, j: (i, j)))
        args.append(residual)
    if ln is not None:
        assert bn == N
        g, b = ln
        in_specs.append(pl.BlockSpec((1, bn), lambda i, j: (0, j)))
        in_specs.append(pl.BlockSpec((1, bn), lambda i, j: (0, j)))
        args.extend([g.reshape(1, N), b.reshape(1, N)])
    kern = functools.partial(
        _mm_kern, nbias=bias is not None, nres=residual is not None,
        act=act, ln=ln is not None)
    return pl.pallas_call(
        kern,
        grid=grid,
        in_specs=in_specs,
        out_specs=pl.BlockSpec((bm, bn), lambda i, j: (i, j)),
        out_shape=jax.ShapeDtypeStruct((M, N), jnp.float32),
        interpret=_INTERPRET,
    )(*args)


def _embed_kern(x_ref, w_ref, xm_ref, wm_ref, pos_ref, out_ref):
    tok = _bdot(x_ref[...], w_ref[...])
    tmp = _bdot(xm_ref[...], wm_ref[...])
    out_ref[...] = tok + tmp + pos_ref[...]


def _embed(xtok, wtok, xm, wm, pos):
    Kdim = xtok.shape[1]
    bm = 512
    grid = (B * T // bm, 1)
    return pl.pallas_call(
        _embed_kern,
        grid=grid,
        in_specs=[
            pl.BlockSpec((bm, Kdim), lambda i, j: (i, 0)),
            pl.BlockSpec((Kdim, D), lambda i, j: (0, 0)),
            pl.BlockSpec((bm, MARK), lambda i, j: (i, 0)),
            pl.BlockSpec((MARK, D), lambda i, j: (0, 0)),
            pl.BlockSpec((bm, D), lambda i, j: (i % (T // bm), 0)),
        ],
        out_specs=pl.BlockSpec((bm, D), lambda i, j: (i, 0)),
        out_shape=jax.ShapeDtypeStruct((B * T, D), jnp.float32),
        interpret=_INTERPRET,
    )(xtok, wtok, xm, wm, pos)


def _hash_kern(qk_ref, rot_ref, out_ref):
    q = qk_ref[...]
    bt = q.shape[1]
    r = _bdot(q.reshape(2 * bt, DH), rot_ref[...])
    r4 = r.reshape(2 * bt, NHASH, NBUCK // 2)
    iota = lax.broadcasted_iota(jnp.int32, r4.shape, 2)
    m1 = jnp.max(r4, axis=-1, keepdims=True)
    i1 = jnp.min(jnp.where(r4 >= m1, iota, NBUCK), axis=-1)
    neg = -r4
    m2 = jnp.max(neg, axis=-1, keepdims=True)
    i2 = jnp.min(jnp.where(neg >= m2, iota, NBUCK), axis=-1)
    bkt = jnp.where(m1[..., 0] >= m2[..., 0], i1, i2 + NBUCK // 2)
    bkt = bkt.reshape(2, bt, NHASH).transpose(0, 2, 1).reshape(2 * NHASH, bt)
    out_ref[...] = bkt


def _hash(qkh, rot2):
    bt = 512
    grid = (BH // 2, T // bt)
    return pl.pallas_call(
        _hash_kern,
        grid=grid,
        in_specs=[
            pl.BlockSpec((2, bt, DH), lambda i, j: (i, j, 0)),
            pl.BlockSpec((DH, NHASH * (NBUCK // 2)), lambda i, j: (0, 0)),
        ],
        out_specs=pl.BlockSpec((2 * NHASH, bt), lambda i, j: (i, j)),
        out_shape=jax.ShapeDtypeStruct((ROWS, T), jnp.int32),
        interpret=_INTERPRET,
    )(qkh.reshape(BH, T, DH), rot2)


def _sc_sort(bkt):
    info = plsc.get_sparse_core_info()
    NC, NS, LANES = info.num_cores, info.num_subcores, info.num_lanes
    NW = NC * NS
    assert ROWS % NW == 0
    reps = ROWS // NW
    steps = T // LANES

    mesh = plsc.VectorSubcoreMesh(core_axis_name="c", subcore_axis_name="s")

    @functools.partial(
        pl.kernel, mesh=mesh,
        out_type=jax.ShapeDtypeStruct((ROWS, T), jnp.int32),
        compiler_params=pltpu.CompilerParams(needs_layout_passes=False),
        scratch_types=[
            pltpu.VMEM((T,), jnp.int32),
            pltpu.VMEM((T,), jnp.int32),
            pltpu.VMEM((NBUCK * LANES,), jnp.int32),
        ],
    )
    def k(bkt_hbm, st_hbm, b_v, st_v, hist_v):
        wid = lax.axis_index("s") * NC + lax.axis_index("c")
        lanes = lax.iota(jnp.int32, LANES)
        ones = jnp.full((LANES,), 1, jnp.int32)
        zeros = jnp.full((LANES,), 0, jnp.int32)
        for rep in range(reps):
            row = wid * reps + rep
            pltpu.sync_copy(bkt_hbm.at[row], b_v)

            def zero_body(i, _):
                hist_v[pl.ds(i * LANES, LANES)] = zeros
                return ()
            lax.fori_loop(0, NBUCK, zero_body, ())

            def p1_body(i, _):
                tvec = lanes * steps + i
                bvec = plsc.load_gather(b_v, [tvec])
                plsc.addupdate_scatter(hist_v, [bvec * LANES + lanes], ones)
                return ()
            lax.fori_loop(0, steps, p1_body, ())

            def pfx_body(i, c):
                row16 = hist_v[pl.ds(i * LANES, LANES)]
                inc = plsc.cumsum(row16)
                tot = jnp.sum(row16)
                hist_v[pl.ds(i * LANES, LANES)] = inc - row16 + c
                return c + tot
            lax.fori_loop(0, NBUCK, pfx_body, jnp.int32(0))

            def p2_body(i, _):
                tvec = lanes * steps + i
                bvec = plsc.load_gather(b_v, [tvec])
                addr = bvec * LANES + lanes
                dst = plsc.load_gather(hist_v, [addr])
                plsc.store_scatter(st_v, [dst], tvec)
                plsc.addupdate_scatter(hist_v, [addr], ones)
                return ()
            lax.fori_loop(0, steps, p2_body, ())

            pltpu.sync_copy(st_v, st_hbm.at[row])

    return k(bkt)


def _sc_gather(qkh, vh, idx):
    NTOT = ROWS * T
    info = plsc.get_sparse_core_info()
    NW = info.num_cores * info.num_subcores
    per_w = NTOT // NW
    CH = 128
    n_ch = per_w // CH

    mesh = plsc.VectorSubcoreMesh(core_axis_name="c", subcore_axis_name="s")

    @functools.partial(
        pl.kernel, mesh=mesh,
        out_type=[jax.ShapeDtypeStruct((NTOT, DH), jnp.float32)] * 2,
        compiler_params=pltpu.CompilerParams(
            needs_layout_passes=False, use_tc_tiling_on_sc=False),
        scratch_types=[
            pltpu.VMEM((CH,), jnp.int32),
            pltpu.VMEM((CH, DH), jnp.float32),
            pltpu.VMEM((CH, DH), jnp.float32),
            pltpu.SemaphoreType.DMA,
            pltpu.SemaphoreType.DMA,
        ],
    )
    def k(qk_hbm, v_hbm, idx_hbm, oq_hbm, ov_hbm, idx_v, bq_v, bv_v, s1, s2):
        wid = lax.axis_index("s") * info.num_cores + lax.axis_index("c")
        base = wid * per_w

        def body(c, _):
            off = base + c * CH
            pltpu.sync_copy(idx_hbm.at[pl.ds(off, CH)], idx_v)
            cp1 = pltpu.async_copy(qk_hbm.at[idx_v], bq_v, s1)
            cp2 = pltpu.async_copy(v_hbm.at[idx_v], bv_v, s2)
            cp1.wait()
            cp2.wait()
            pltpu.sync_copy(bq_v, oq_hbm.at[pl.ds(off, CH)])
            pltpu.sync_copy(bv_v, ov_hbm.at[pl.ds(off, CH)])
            return ()
        lax.fori_loop(0, n_ch, body, ())

    return k(qkh, vh, idx)


def _sc_scatter(so_ext, idx):
    NTOT, EW = so_ext.shape
    info = plsc.get_sparse_core_info()
    NW = info.num_cores * info.num_subcores
    per_w = NTOT // NW
    CH = 128
    n_ch = per_w // CH

    mesh = plsc.VectorSubcoreMesh(core_axis_name="c", subcore_axis_name="s")

    @functools.partial(
        pl.kernel, mesh=mesh,
        out_type=jax.ShapeDtypeStruct((NTOT, EW), jnp.float32),
        compiler_params=pltpu.CompilerParams(
            needs_layout_passes=False, use_tc_tiling_on_sc=False),
        scratch_types=[
            pltpu.VMEM((CH,), jnp.int32),
            pltpu.VMEM((CH, EW), jnp.float32),
            pltpu.SemaphoreType.DMA,
        ],
    )
    def k(src_hbm, idx_hbm, out_hbm, idx_v, rows_v, sem):
        wid = lax.axis_index("s") * info.num_cores + lax.axis_index("c")
        base = wid * per_w

        def body(c, _):
            off = base + c * CH
            pltpu.sync_copy(idx_hbm.at[pl.ds(off, CH)], idx_v)
            pltpu.sync_copy(src_hbm.at[pl.ds(off, CH)], rows_v)
            pltpu.async_copy(rows_v, out_hbm.at[idx_v], sem).wait()
            return ()
        lax.fori_loop(0, n_ch, body, ())

    return k(so_ext, idx)


EW = 80


def _attn_kern(sqk_ref, sv_ref, st_ref, out_ref):
    q = sqk_ref[0]
    v = sv_ref[0]
    t3 = st_ref[0]
    nrm = jnp.sqrt(jnp.sum(q * q, axis=-1, keepdims=True))
    kn = q / jnp.clip(nrm, 1e-12, None)
    pk = jnp.concatenate([kn[-BK:], kn[:-BK]], axis=0)
    pv = jnp.concatenate([v[-BK:], v[:-BK]], axis=0)
    pt3 = jnp.concatenate([t3[-1:], t3[:-1]], axis=0)
    scale = DH ** -0.5
    SUB = 2048
    for sub in range(S // SUB):
        sl = slice(sub * SUB, (sub + 1) * SUB)
        nc = SUB // BK
        slc = slice(sub * nc, (sub + 1) * nc)
        bq = q[sl].reshape(nc, BK, DH)
        bk8 = jnp.concatenate(
            [kn[sl].reshape(nc, BK, DH), pk[sl].reshape(nc, BK, DH)], axis=1)
        bv8 = jnp.concatenate(
            [v[sl].reshape(nc, BK, DH), pv[sl].reshape(nc, BK, DH)], axis=1)
        tq = t3[slc]
        tk = jnp.concatenate([tq, pt3[slc]], axis=1)
        dots = jnp.sum(_b16(bq)[:, :, None, :] * _b16(bk8)[:, None, :, :],
                       axis=-1) * scale
        eq = tq[:, :, None] == tk[:, None, :]
        dots = jnp.where(eq, -5e4, dots)
        m = jnp.max(dots, axis=-1, keepdims=True)
        ssum = jnp.sum(jnp.exp(dots - m), axis=-1, keepdims=True)
        lse = m + jnp.log(ssum)
        probs = jnp.exp(dots - lse)
        bo = jnp.sum(_b16(probs)[:, :, :, None] * _b16(bv8)[:, None, :, :],
                     axis=2)
        ocat = jnp.concatenate(
            [bo.reshape(SUB, DH), lse.reshape(SUB, 1),
             jnp.zeros((SUB, EW - DH - 1), jnp.float32)], axis=-1)
        out_ref[0, pl.ds(sub * SUB, SUB), :] = ocat


def _attn(sqk, sv, st):
    grid = (BH,)
    return pl.pallas_call(
        _attn_kern,
        grid=grid,
        in_specs=[
            pl.BlockSpec((1, S, DH), lambda i: (i, 0, 0)),
            pl.BlockSpec((1, S, DH), lambda i: (i, 0, 0)),
            pl.BlockSpec((1, S // BK, BK), lambda i: (i, 0, 0)),
        ],
        out_specs=pl.BlockSpec((1, S, EW), lambda i: (i, 0, 0)),
        out_shape=jax.ShapeDtypeStruct((BH, S, EW), jnp.float32),
        interpret=_INTERPRET,
    )(sqk, sv, st)


def _mix_kern(o_ref, out_ref):
    o = o_ref[0]
    a = o[:, :, :DH]
    l = o[:, :, DH]
    m = jnp.max(l, axis=0, keepdims=True)
    lsm = m + jnp.log(jnp.sum(jnp.exp(l - m), axis=0, keepdims=True))
    w = jnp.exp(l - lsm)
    out_ref[0] = jnp.sum(a * w[:, :, None], axis=0)


def _mix(o_all):
    bt = 512
    grid = (BH, T // bt)
    return pl.pallas_call(
        _mix_kern,
        grid=grid,
        in_specs=[pl.BlockSpec((1, NHASH, bt, EW), lambda i, j: (i, 0, j, 0))],
        out_specs=pl.BlockSpec((1, bt, DH), lambda i, j: (i, j, 0)),
        out_shape=jax.ShapeDtypeStruct((BH, T, DH), jnp.float32),
        interpret=_INTERPRET,
    )(o_all)


def _ln_kern(x_ref, g_ref, b_ref, out_ref):
    x = x_ref[...]
    m = jnp.mean(x, axis=-1, keepdims=True)
    v = jnp.mean((x - m) ** 2, axis=-1, keepdims=True)
    out_ref[...] = (x - m) / jnp.sqrt(v + 1e-5) * g_ref[...] + b_ref[...]


def _ln(x, g, b):
    M, N = x.shape
    bm = min(512, M)
    grid = (M // bm,)
    return pl.pallas_call(
        _ln_kern,
        grid=grid,
        in_specs=[
            pl.BlockSpec((bm, N), lambda i: (i, 0)),
            pl.BlockSpec((1, N), lambda i: (0, 0)),
            pl.BlockSpec((1, N), lambda i: (0, 0)),
        ],
        out_specs=pl.BlockSpec((bm, N), lambda i: (i, 0)),
        out_shape=jax.ShapeDtypeStruct((M, N), jnp.float32),
        interpret=_INTERPRET,
    )(x, g.reshape(1, N), b.reshape(1, N))


_USE_SC_SORT = True
_USE_SC_GATHER = True
_USE_SC_SCATTER = True


def _buckets_xla(xb, qk_w, rot):
    qk = (xb @ qk_w.T).reshape(B, T, H, DH).transpose(0, 2, 1, 3)
    rotated = jnp.einsum('btf,fhi->bhti', qk.reshape(BH, T, DH), rot)
    rotated = jnp.concatenate([rotated, -rotated], axis=-1)
    return jnp.argmax(rotated, axis=-1).astype(jnp.int32).reshape(ROWS, T)


def _lsh_attention(x2d, xb, qk_w, v_w, out_w, out_b, rot):
    qk2d = _mm(x2d, qk_w.T)
    v2d = _mm(x2d, v_w.T)
    qkh = qk2d.reshape(B, T, H, DH).transpose(0, 2, 1, 3).reshape(BH * T, DH)
    vh = v2d.reshape(B, T, H, DH).transpose(0, 2, 1, 3).reshape(BH * T, DH)
    bkt = _buckets_xla(xb, qk_w, rot)

    if _USE_SC_SORT:
        st = _sc_sort(bkt)
    else:
        key = bkt * T + jnp.arange(T, dtype=jnp.int32)[None, :]
        st = jnp.argsort(key, axis=-1).astype(jnp.int32)

    row_ids = jnp.arange(ROWS, dtype=jnp.int32)[:, None]
    gidx = (st + (row_ids // NHASH) * T).reshape(-1)
    if _USE_SC_GATHER:
        sqk, sv = _sc_gather(qkh, vh, gidx)
    else:
        sqk = jnp.take(qkh, gidx, axis=0)
        sv = jnp.take(vh, gidx, axis=0)

    so_ext = _attn(sqk.reshape(BH, S, DH), sv.reshape(BH, S, DH),
                   st.reshape(BH, S // BK, BK))

    sidx = (st + row_ids * T).reshape(-1)
    so_flat = so_ext.reshape(ROWS * T, EW)
    if _USE_SC_SCATTER:
        o_all = _sc_scatter(so_flat, sidx)
    else:
        o_all = jnp.zeros_like(so_flat).at[sidx].set(so_flat)

    mixed = _mix(o_all.reshape(BH, NHASH, T, EW))
    m2d = mixed.reshape(B, H, T, DH).transpose(0, 2, 1, 3).reshape(B * T, D)
    return _mm(m2d, out_w.T, bias=out_b, residual=x2d)


def _layer_tail(xsum, c1_w, c1_b, c2_w, c2_b, n1_g, n1_b, n2_g, n2_b):
    x1 = _ln(xsum, n1_g, n1_b)
    y1 = _mm(x1, c1_w.T, bias=c1_b, act="gelu")
    return _mm(y1, c2_w.T, bias=c2_b, residual=x1, ln=(n2_g, n2_b))


def _pos_emb():
    import numpy as np
    pos = np.arange(T)[:, None].astype(np.float32)
    div = np.exp(np.arange(0, D, 2).astype(np.float32) * -(np.log(10000.0) / D))
    pe = np.zeros((T, D), np.float32)
    pe[:, 0::2] = np.sin(pos * div)
    pe[:, 1::2] = np.cos(pos * div)
    return jnp.asarray(pe)


def kernel(x_enc, x_mark_enc, x_dec, x_mark_dec, token_w, time_w,
           qk_w0, v_w0, out_w0, out_b0, rot0, c1_w0, c1_b0, c2_w0, c2_b0,
           n1_g0, n1_b0, n2_g0, n2_b0,
           qk_w1, v_w1, out_w1, out_b1, rot1, c1_w1, c1_b1, c2_w1, c2_b1,
           n1_g1, n1_b1, n2_g1, n2_b1,
           norm_g, norm_b, proj_w, proj_b):
    xe = jnp.concatenate(
        [x_enc, jnp.zeros((B, 1, ENC_IN), jnp.float32)], axis=1)
    xm = jnp.concatenate([x_mark_enc, x_mark_dec[:, -1:, :]], axis=1)
    left = jnp.roll(xe, 1, axis=1)
    right = jnp.roll(xe, -1, axis=1)
    zc = jnp.zeros((B, T, 128 - ENC_IN), jnp.float32)
    xtok = jnp.concatenate([left, zc, xe, zc, right, zc], axis=-1)
    zw = jnp.zeros((128 - ENC_IN, D), jnp.float32)
    wtok = jnp.concatenate(
        [token_w[:, :, 0].T, zw, token_w[:, :, 1].T, zw,
         token_w[:, :, 2].T, zw], axis=0)
    x = _embed(xtok.reshape(B * T, 384), wtok,
               xm.reshape(B * T, MARK), time_w.T, _pos_emb())

    xpw = jnp.pad(xe, ((0, 0), (1, 1), (0, 0)), mode='wrap')
    tok = jax.lax.conv_general_dilated(
        xpw, token_w, (1,), 'VALID', dimension_numbers=('NWC', 'OIW', 'NWC'))
    xb0 = (tok + xm @ time_w.T + _pos_emb()[None]).reshape(B * T, D)

    xsum0 = _lsh_attention(x, xb0, qk_w0, v_w0, out_w0, out_b0, rot0)
    x = _layer_tail(xsum0, c1_w0, c1_b0, c2_w0, c2_b0,
                    n1_g0, n1_b0, n2_g0, n2_b0)

    xsum1 = _lsh_attention(x, x, qk_w1, v_w1, out_w1, out_b1, rot1)
    x = _layer_tail(xsum1, c1_w1, c1_b1, c2_w1, c2_b1,
                    n1_g1, n1_b1, n2_g1, n2_b1)

    xtail = x.reshape(B, T, D)[:, -8:, :].reshape(B * 8, D)
    xn = _ln(xtail, norm_g, norm_b)
    out = _mm(xn, proj_w.T, bias=proj_b, bm=16, bn=C_OUT)
    return out.reshape(B, 8, C_OUT)[:, -1:, :]
